# Initial kernel scaffold; baseline (speedup 1.0000x reference)
import jax
import jax.numpy as jnp
from jax.experimental import pallas as pl


def kernel(x, edge_index, edge_weight, batch, bn_gamma, bn_beta, W, bias):
    raise NotImplementedError("write your pallas kernel here")


# speedup vs baseline: 7.1486x; 7.1486x over previous
import functools

import jax
import jax.numpy as jnp
from jax import lax
from jax.experimental import pallas as pl
from jax.experimental.pallas import tpu as pltpu
from jax.experimental.pallas import tpu_sc as plsc

N = 10000
E = 160000
D = 256
T = 8
G = 128
NUM_ROUTES = 3

NC = 2
NS = 16
NW = NC * NS
DH = D // 2

CHUNK = 128
NCHUNK = 80
EPT = NCHUNK * CHUNK
EPAD = NS * EPT
EPW = EPAD // NW
EPWR = EPW // CHUNK
NR1 = 80

NP = 10240
RPT = NP // NS

N2 = 10240
NB = 512
NBLK = N2 // NB

def _sc1_body(src_hbm, dst_hbm, ew_hbm, outd_hbm, outs_hbm,
              src_v, dst_v, ew_v, accd_v, accs_v):
    c = lax.axis_index("c")
    s = lax.axis_index("s")
    wid = s * NC + c
    pltpu.sync_copy(src_hbm.at[wid], src_v)
    pltpu.sync_copy(dst_hbm.at[wid], dst_v)
    pltpu.sync_copy(ew_hbm.at[wid], ew_v)

    zv = jnp.zeros((16,), jnp.float32)

    def zero_body(r, carry):
        for k in range(CHUNK // 16):
            accd_v[r, pl.ds(k * 16, 16)] = zv
            accs_v[r, pl.ds(k * 16, 16)] = zv
        return carry

    lax.fori_loop(0, NR1, zero_body, 0)

    lanes = lax.iota(jnp.int32, 16)

    def edge_body(r, carry):
        for g in range(CHUNK // 16):
            w16 = ew_v[r, pl.ds(g * 16, 16)]
            d16 = dst_v[r, pl.ds(g * 16, 16)]
            s16 = src_v[r, pl.ds(g * 16, 16)]
            for k in range(16):
                w = w16[k]
                d = d16[k]
                cd = ((d % CHUNK) // 16) * 16
                vald = jnp.where(lanes == d % 16, w, 0.0)
                plsc.addupdate(accd_v.at[d // CHUNK, pl.ds(cd, 16)], vald)
                sn = s16[k]
                cs = ((sn % CHUNK) // 16) * 16
                vals = jnp.where(lanes == sn % 16, w, 0.0)
                plsc.addupdate(accs_v.at[sn // CHUNK, pl.ds(cs, 16)], vals)
        return carry

    lax.fori_loop(0, EPWR, edge_body, 0)

    pltpu.sync_copy(accd_v, outd_hbm.at[wid])
    pltpu.sync_copy(accs_v, outs_hbm.at[wid])


def _sc2_body(xlo_hbm, xhi_hbm, srcs_hbm, dsts_hbm, ews_hbm, out_hbm,
              src_v, dst_v, ew_v, rows_v, acc_sh, sem):
    c = lax.axis_index("c")
    s = lax.axis_index("s")

    pltpu.sync_copy(srcs_hbm.at[s], src_v)
    pltpu.sync_copy(dsts_hbm.at[s], dst_v)
    pltpu.sync_copy(ews_hbm.at[s], ew_v)

    zv = jnp.zeros((16,), jnp.float32)

    def zrow(r, carry):
        for k in range(DH // 16):
            rows_v[r, pl.ds(k * 16, 16)] = zv
        return carry

    lax.fori_loop(0, CHUNK, zrow, 0)
    for k in range(RPT // CHUNK):
        pltpu.sync_copy(rows_v, acc_sh.at[pl.ds(s * RPT + k * CHUNK, CHUNK)])
    plsc.subcore_barrier()

    def chunk_body(j, carry):
        sidx = src_v.at[j]

        @pl.when(c == 0)
        def _():
            pltpu.async_copy(xlo_hbm.at[sidx], rows_v, sem).wait()

        @pl.when(c == 1)
        def _():
            pltpu.async_copy(xhi_hbm.at[sidx], rows_v, sem).wait()

        def scale_rows(g, rcarry):
            w16 = ew_v[j, pl.ds(g * 16, 16)]
            for k in range(16):
                w = w16[k]
                r = g * 16 + k
                for k2 in range(DH // 16):
                    rows_v[r, pl.ds(k2 * 16, 16)] = (
                        rows_v[r, pl.ds(k2 * 16, 16)] * w)
            return rcarry

        lax.fori_loop(0, CHUNK // 16, scale_rows, 0)

        pltpu.sync_copy(rows_v, acc_sh.at[dst_v.at[j]], add=True)
        return carry

    lax.fori_loop(0, NCHUNK, chunk_body, 0)
    plsc.subcore_barrier()

    for k in range(RPT // CHUNK):
        r0 = s * RPT + k * CHUNK
        pltpu.sync_copy(acc_sh.at[pl.ds(r0, CHUNK)], out_hbm.at[c].at[pl.ds(r0, CHUNK)])


@functools.lru_cache(maxsize=None)
def _sc_kernels():
    mesh = plsc.VectorSubcoreMesh(core_axis_name="c", subcore_axis_name="s",
                                  num_cores=NC, num_subcores=NS)
    sc1 = pl.kernel(
        _sc1_body,
        mesh=mesh,
        out_type=(
            jax.ShapeDtypeStruct((NW, NR1, CHUNK), jnp.float32),
            jax.ShapeDtypeStruct((NW, NR1, CHUNK), jnp.float32),
        ),
        scratch_types=[
            pltpu.VMEM((EPWR, CHUNK), jnp.int32),
            pltpu.VMEM((EPWR, CHUNK), jnp.int32),
            pltpu.VMEM((EPWR, CHUNK), jnp.float32),
            pltpu.VMEM((NR1, CHUNK), jnp.float32),
            pltpu.VMEM((NR1, CHUNK), jnp.float32),
        ],
    )
    sc2 = pl.kernel(
        _sc2_body,
        mesh=mesh,
        out_type=jax.ShapeDtypeStruct((NC, NP, DH), jnp.float32),
        scratch_types=[
            pltpu.VMEM((NCHUNK, CHUNK), jnp.int32),
            pltpu.VMEM((NCHUNK, CHUNK), jnp.int32),
            pltpu.VMEM((NCHUNK, CHUNK), jnp.float32),
            pltpu.VMEM((CHUNK, DH), jnp.float32),
            pltpu.VMEM_SHARED((NP, DH), jnp.float32),
            pltpu.SemaphoreType.DMA,
        ],
    )
    return sc1, sc2


def _rb(a):
    return a.astype(jnp.bfloat16).astype(jnp.float32)


def _split3(a):
    a0 = _rb(a)
    r = a - a0
    a1 = _rb(r)
    return a0, a1, r - a1


def _split2(a):
    a0 = _rb(a)
    return a0, a - a0


_CN = (((0,), (0,)), ((), ()))
_CD = (((1,), (1,)), ((), ()))
_HP = lax.Precision.HIGHEST


def _dotx3(a, b, dims):
    a0, a1, a2 = _split3(a)
    acc = lax.dot_general(a0, b, dims, preferred_element_type=jnp.float32,
                          precision=_HP)
    acc += lax.dot_general(a1, b, dims, preferred_element_type=jnp.float32,
                           precision=_HP)
    acc += lax.dot_general(a2, b, dims, preferred_element_type=jnp.float32,
                           precision=_HP)
    return acc


def _dotx3b(a, b, dims):
    b0, b1, b2 = _split3(b)
    acc = lax.dot_general(a, b0, dims, preferred_element_type=jnp.float32,
                          precision=_HP)
    acc += lax.dot_general(a, b1, dims, preferred_element_type=jnp.float32,
                           precision=_HP)
    acc += lax.dot_general(a, b2, dims, preferred_element_type=jnp.float32,
                           precision=_HP)
    return acc


def _dotxx(a, b, dims):
    a0, a1 = _split2(a)
    b0, b1 = _split2(b)
    acc = lax.dot_general(a0, b0, dims, preferred_element_type=jnp.float32,
                          precision=_HP)
    acc += lax.dot_general(a0, b1, dims, preferred_element_type=jnp.float32,
                           precision=_HP)
    acc += lax.dot_general(a1, b0, dims, preferred_element_type=jnp.float32,
                           precision=_HP)
    return acc


NBA = 1000


def _tca1_body(x_ref, s1_ref, s2_ref):
    i = pl.program_id(0)

    @pl.when(i == 0)
    def _():
        s1_ref[...] = jnp.zeros_like(s1_ref)
        s2_ref[...] = jnp.zeros_like(s2_ref)

    x = x_ref[...]
    s1_ref[...] += jnp.sum(x, axis=0, keepdims=True)
    s2_ref[...] += jnp.sum(x * x, axis=0, keepdims=True)


def _tca1(x):
    return pl.pallas_call(
        _tca1_body,
        grid=(N // NBA,),
        in_specs=[pl.BlockSpec((NBA, D), lambda i: (i, 0))],
        out_specs=(pl.BlockSpec((1, D), lambda i: (0, 0)),
                   pl.BlockSpec((1, D), lambda i: (0, 0))),
        out_shape=(jax.ShapeDtypeStruct((1, D), jnp.float32),
                   jax.ShapeDtypeStruct((1, D), jnp.float32)),
    )(x)


def _tca2_body(x_ref, part_ref, gamma_ref, beta_ref, batch_ref, m_ref, v_ref,
               xbn_ref, xs_ref, dinv_ref, ndeg_ref, xsum_ref, counts_ref):
    i = pl.program_id(0)
    xbn = ((x_ref[...] - m_ref[...]) / jnp.sqrt(v_ref[...] + 1e-5)
           * gamma_ref[...] + beta_ref[...])
    xbn_ref[...] = xbn

    part = part_ref[...]
    deg = jnp.sum(part[:, :NW], axis=1, keepdims=True) + 1.0
    ndeg = jnp.sum(part[:, NW:], axis=1, keepdims=True)
    dinv = lax.rsqrt(deg)
    dinv_ref[...] = dinv
    ndeg_ref[...] = ndeg
    xq = xbn.astype(jnp.bfloat16).astype(jnp.float32)
    xs_ref[...] = xq * dinv

    oh = (batch_ref[...] == lax.broadcasted_iota(jnp.int32, (1, G), 1))
    oh = oh.astype(jnp.float32)

    @pl.when(i == 0)
    def _():
        xsum_ref[...] = jnp.zeros_like(xsum_ref)
        counts_ref[...] = jnp.zeros_like(counts_ref)

    xsum_ref[...] += _dotx3b(oh, xbn, _CN)
    counts_ref[...] += lax.dot_general(oh, jnp.ones((NBA, 1), jnp.float32),
                                       (((0,), (0,)), ((), ())),
                                       preferred_element_type=jnp.float32,
                                       precision=lax.Precision.HIGHEST)


def _tca2(x, part_t, gamma, beta, batch_col, mean, var):
    return pl.pallas_call(
        _tca2_body,
        grid=(N // NBA,),
        in_specs=[
            pl.BlockSpec((NBA, D), lambda i: (i, 0)),
            pl.BlockSpec((NBA, 2 * NW), lambda i: (i, 0)),
            pl.BlockSpec((1, D), lambda i: (0, 0)),
            pl.BlockSpec((1, D), lambda i: (0, 0)),
            pl.BlockSpec((NBA, 1), lambda i: (i, 0)),
            pl.BlockSpec((1, D), lambda i: (0, 0)),
            pl.BlockSpec((1, D), lambda i: (0, 0)),
        ],
        out_specs=(
            pl.BlockSpec((NBA, D), lambda i: (i, 0)),
            pl.BlockSpec((NBA, D), lambda i: (i, 0)),
            pl.BlockSpec((NBA, 1), lambda i: (i, 0)),
            pl.BlockSpec((NBA, 1), lambda i: (i, 0)),
            pl.BlockSpec((G, D), lambda i: (0, 0)),
            pl.BlockSpec((G, 1), lambda i: (0, 0)),
        ),
        out_shape=(
            jax.ShapeDtypeStruct((N, D), jnp.float32),
            jax.ShapeDtypeStruct((N, D), jnp.float32),
            jax.ShapeDtypeStruct((N, 1), jnp.float32),
            jax.ShapeDtypeStruct((N, 1), jnp.float32),
            jax.ShapeDtypeStruct((G, D), jnp.float32),
            jax.ShapeDtypeStruct((G, 1), jnp.float32),
        ),
    )(x, part_t, gamma, beta, batch_col, mean, var)


def _ty_body(agg_ref, xbn_ref, dinv_ref, y_ref):
    dinv = dinv_ref[...]
    xq = xbn_ref[...].astype(jnp.bfloat16).astype(jnp.float32)
    y_ref[...] = dinv * agg_ref[...] + dinv * dinv * xq


def _ty(agg, xbn, dinv):
    return pl.pallas_call(
        _ty_body,
        grid=(NBLK,),
        in_specs=[
            pl.BlockSpec((NB, D), lambda i: (i, 0)),
            pl.BlockSpec((NB, D), lambda i: (i, 0)),
            pl.BlockSpec((NB, 1), lambda i: (i, 0)),
        ],
        out_specs=pl.BlockSpec((NB, D), lambda i: (i, 0)),
        out_shape=jax.ShapeDtypeStruct((N2, D), jnp.float32),
    )(agg, xbn, dinv)


def _softmax_rows(b):
    bm = jnp.max(b, axis=1, keepdims=True)
    e = jnp.exp(b - bm)
    return e / jnp.sum(e, axis=1, keepdims=True)


def _onehot(batch_blk):
    oh = batch_blk == lax.broadcasted_iota(jnp.int32, (1, G), 1)
    return oh.astype(jnp.float32)


def _kz_body(y_ref, b_ref, batch_ref, z_ref, csum_ref):
    i = pl.program_id(0)

    @pl.when(i == 0)
    def _():
        z_ref[...] = jnp.zeros_like(z_ref)
        csum_ref[...] = jnp.zeros_like(csum_ref)

    y = y_ref[...]
    c = _softmax_rows(b_ref[...])
    oh = _onehot(batch_ref[...])
    cy = jnp.concatenate([c[:, t:t + 1] * y for t in range(T)], axis=1)
    z_ref[...] += _dotx3b(oh, cy, _CN)
    csum_ref[...] += _dotx3b(oh, c, _CN)


def _kz(y, b, batch_col):
    return pl.pallas_call(
        _kz_body,
        grid=(NBLK,),
        in_specs=[
            pl.BlockSpec((NB, D), lambda i: (i, 0)),
            pl.BlockSpec((NB, T), lambda i: (i, 0)),
            pl.BlockSpec((NB, 1), lambda i: (i, 0)),
        ],
        out_specs=(
            pl.BlockSpec((G, T * D), lambda i: (0, 0)),
            pl.BlockSpec((G, T), lambda i: (0, 0)),
        ),
        out_shape=(
            jax.ShapeDtypeStruct((G, T * D), jnp.float32),
            jax.ShapeDtypeStruct((G, T), jnp.float32),
        ),
    )(y, b, batch_col)


def _squash_rows(s):
    n2 = jnp.sum(s * s, axis=1, keepdims=True)
    return (n2 / (1.0 + n2)) * s * lax.rsqrt(n2 + 1e-8)


def _ksmall_body(z_ref, csum_ref, w_ref, bias_ref, v_ref):
    for t in range(T):
        wq = _rb(w_ref[t])
        zt = z_ref[:, t * D:(t + 1) * D]
        st = (_dotx3(zt, wq, (((1,), (0,)), ((), ())))
              + csum_ref[:, t:t + 1] * bias_ref[t][None, :])
        v_ref[:, t * D:(t + 1) * D] = _squash_rows(st)


def _ksmall(z, csum, w, bias):
    return pl.pallas_call(
        _ksmall_body,
        out_shape=jax.ShapeDtypeStruct((G, T * D), jnp.float32),
    )(z, csum, w, bias)


def _kb_body(y_ref, b_ref, batch_ref, w_ref, bias_ref, v_ref, bnew_ref):
    y = y_ref[...]
    oh = _onehot(batch_ref[...])
    cols = []
    for t in range(T):
        ut = (_dotx3(y, _rb(w_ref[t]), (((1,), (0,)), ((), ())))
              + bias_ref[t][None, :])
        vsel = _dotx3b(oh, v_ref[:, t * D:(t + 1) * D],
                       (((1,), (0,)), ((), ())))
        cols.append(jnp.sum(ut * vsel, axis=1, keepdims=True))
    bnew_ref[...] = b_ref[...] + jnp.concatenate(cols, axis=1)


def _kb(y, b, batch_col, w, bias, v):
    return pl.pallas_call(
        _kb_body,
        grid=(NBLK,),
        in_specs=[
            pl.BlockSpec((NB, D), lambda i: (i, 0)),
            pl.BlockSpec((NB, T), lambda i: (i, 0)),
            pl.BlockSpec((NB, 1), lambda i: (i, 0)),
            pl.BlockSpec((T, D, D), lambda i: (0, 0, 0)),
            pl.BlockSpec((T, D), lambda i: (0, 0)),
            pl.BlockSpec((G, T * D), lambda i: (0, 0)),
        ],
        out_specs=pl.BlockSpec((NB, T), lambda i: (i, 0)),
        out_shape=jax.ShapeDtypeStruct((N2, T), jnp.float32),
    )(y, b, batch_col, w, bias, v)


def _kfinal_body(z_ref, csum_ref, w_ref, bias_ref, xsum_ref, counts_ref,
                 out_ref):
    xmean = xsum_ref[...] / jnp.maximum(counts_ref[...], 1.0)
    for t in range(T):
        zt = z_ref[:, t * D:(t + 1) * D]
        st = (_dotx3(zt, _rb(w_ref[t]), (((1,), (0,)), ((), ())))
              + csum_ref[:, t:t + 1] * bias_ref[t][None, :]
              + xmean)
        vt = _squash_rows(st)
        rec = 1.0 / jnp.abs(vt)
        out_ref[:, t:t + 1] = 1.0 / jnp.sum(rec, axis=1, keepdims=True)


def _kfinal(z, csum, w, bias, xsum, counts):
    return pl.pallas_call(
        _kfinal_body,
        out_shape=jax.ShapeDtypeStruct((G, T), jnp.float32),
    )(z, csum, w, bias, xsum, counts)


def kernel(x, edge_index, edge_weight, batch, bn_gamma, bn_beta, W, bias):
    src = edge_index[0]
    dst = edge_index[1]
    pad = EPAD - E
    src_p = jnp.pad(src, (0, pad))
    dst_p = jnp.pad(dst, (0, pad))
    ew_p = jnp.pad(edge_weight, (0, pad))

    sc1, sc2 = _sc_kernels()
    partd, parts = sc1(src_p.reshape(NW, EPWR, CHUNK),
                       dst_p.reshape(NW, EPWR, CHUNK),
                       ew_p.reshape(NW, EPWR, CHUNK))
    partd = partd.reshape(NW, NR1 * CHUNK)[:, :N]
    parts = parts.reshape(NW, NR1 * CHUNK)[:, :N]
    part_t = jnp.concatenate([partd.T, parts.T], axis=1)

    batch_col = batch.reshape(N, 1)
    mean = jnp.mean(x, axis=0).reshape(1, D)
    var = jnp.var(x, axis=0).reshape(1, D)
    xbn, xs, dinv, ndeg, xsum, counts = _tca2(
        x, part_t, bn_gamma.reshape(1, D), bn_beta.reshape(1, D),
        batch_col, mean, var)

    agg2 = sc2(xs[:, :DH], xs[:, DH:],
               src_p.reshape(NS, NCHUNK, CHUNK),
               dst_p.reshape(NS, NCHUNK, CHUNK),
               ew_p.reshape(NS, NCHUNK, CHUNK))
    agg = jnp.concatenate([agg2[0], agg2[1]], axis=1)

    padn = N2 - N
    agg_p = agg[:N2]
    xbn_p = jnp.pad(xbn, ((0, padn), (0, 0)))
    dinv_p = jnp.pad(dinv, ((0, padn), (0, 0)))
    batch_p = jnp.pad(batch_col, ((0, padn), (0, 0)), constant_values=G)

    y = _ty(agg_p, xbn_p, dinv_p)
    b = jnp.broadcast_to(jnp.pad(ndeg, ((0, padn), (0, 0))), (N2, T))

    for _ in range(NUM_ROUTES - 1):
        z, csum = _kz(y, b, batch_p)
        v = _ksmall(z, csum, W, bias)
        b = _kb(y, b, batch_p, W, bias, v)

    z, csum = _kz(y, b, batch_p)
    return _kfinal(z, csum, W, bias, xsum, counts)
